# baseline (speedup 1.0000x reference)
import functools

import jax
import jax.numpy as jnp
from jax import lax
from jax.experimental import pallas as pl
from jax.experimental.pallas import tpu as pltpu
from jax.experimental.pallas import tpu_sc as plsc

_B = 16
_S = 64
_R = _S - 1
_D = 1024
_CH = 32
_NW = 32
_CHUNKS = _B * _R * 2
_PER_W = _CHUNKS // _NW


_CHW = _CH * _D


def _chunk_rows(g):
    task = g // 2
    half = g - task * 2
    b = task // _R
    i = task - b * _R
    src = b * (_S * _S) + i * (_S + 1) + 1 + half * _CH
    dst = b * (_R * _S) + i * _S + half * _CH
    return src * _D, dst * _D


def kernel(arr):
    B, S2, D = arr.shape
    src1 = arr.reshape(B * S2 * D)

    mesh = plsc.VectorSubcoreMesh(core_axis_name="c", subcore_axis_name="s")

    @functools.partial(
        pl.kernel,
        mesh=mesh,
        out_type=jax.ShapeDtypeStruct((_B * _R * _S * _D,), arr.dtype),
        scratch_types=[
            pltpu.VMEM((_CHW,), jnp.float32),
            pltpu.VMEM((_CHW,), jnp.float32),
            pltpu.SemaphoreType.DMA,
            pltpu.SemaphoreType.DMA,
            pltpu.SemaphoreType.DMA,
            pltpu.SemaphoreType.DMA,
        ],
    )
    def copy_kernel(in_hbm, out_hbm, buf0, buf1, g0, g1, s0, s1):
        wid = lax.axis_index("s") * 2 + lax.axis_index("c")
        bufs = (buf0, buf1)
        gsems = (g0, g1)
        ssems = (s0, s1)

        def gather(j, ph):
            src, _ = _chunk_rows(wid + j * _NW)
            pltpu.async_copy(in_hbm.at[pl.ds(src, _CHW)], bufs[ph], gsems[ph])

        def scatter(j, ph):
            _, dst = _chunk_rows(wid + j * _NW)
            pltpu.async_copy(bufs[ph], out_hbm.at[pl.ds(dst, _CHW)], ssems[ph])

        gather(0, 0)
        for j in range(_PER_W):
            ph = j % 2
            nxt = (j + 1) % 2
            if j + 1 < _PER_W:
                if j + 1 >= 2:
                    pltpu.make_async_copy(
                        bufs[nxt], out_hbm.at[pl.ds(0, _CHW)], ssems[nxt]
                    ).wait()
                gather(j + 1, nxt)
            pltpu.make_async_copy(
                in_hbm.at[pl.ds(0, _CHW)], bufs[ph], gsems[ph]
            ).wait()
            scatter(j, ph)
        pltpu.make_async_copy(
            bufs[(_PER_W - 2) % 2], out_hbm.at[pl.ds(0, _CHW)], ssems[(_PER_W - 2) % 2]
        ).wait()
        pltpu.make_async_copy(
            bufs[(_PER_W - 1) % 2], out_hbm.at[pl.ds(0, _CHW)], ssems[(_PER_W - 1) % 2]
        ).wait()

    out1 = copy_kernel(src1)
    return out1.reshape(B, _R * _S, D)
